# Initial kernel scaffold; baseline (speedup 1.0000x reference)
import jax
import jax.numpy as jnp
from jax.experimental import pallas as pl


def kernel(token_ids, table):
    raise NotImplementedError("write your pallas kernel here")


# speedup vs baseline: 4.8016x; 4.8016x over previous
import functools

import jax
import jax.numpy as jnp
from jax import lax
from jax.experimental import pallas as pl
from jax.experimental.pallas import tpu as pltpu
from jax.experimental.pallas import tpu_sc as plsc

EMB_D = 32
L = 128
CR = 8
NW = 32


@functools.cache
def _make_sc_gather(n_rows_idx: int):
    assert n_rows_idx % (NW * CR) == 0
    rows_per_w = n_rows_idx // NW
    chunks = rows_per_w // CR
    mesh = plsc.VectorSubcoreMesh(core_axis_name="c", subcore_axis_name="s")

    @functools.partial(
        pl.kernel,
        mesh=mesh,
        out_type=jax.ShapeDtypeStruct((n_rows_idx * L, EMB_D), jnp.float32),
        scratch_types=[
            pltpu.VMEM((CR, L), jnp.int32),
            pltpu.VMEM((CR * L, EMB_D), jnp.float32),
            pltpu.SemaphoreType.DMA,
        ],
        compiler_params=pltpu.CompilerParams(use_tc_tiling_on_sc=False),
    )
    def k(idx_hbm, table_hbm, out_hbm, idx_v, rows_v, sem):
        wid = lax.axis_index("s") * 2 + lax.axis_index("c")
        row_base = wid * rows_per_w

        def body(g, carry):
            r0 = row_base + g * CR
            pltpu.sync_copy(idx_hbm.at[pl.ds(r0, CR), :], idx_v)
            copies = [
                pltpu.async_copy(
                    table_hbm.at[idx_v.at[j]],
                    rows_v.at[pl.ds(j * L, L), :],
                    sem,
                )
                for j in range(CR)
            ]
            for c in copies:
                c.wait()
            pltpu.sync_copy(rows_v, out_hbm.at[pl.ds(r0 * L, CR * L), :])
            return carry

        lax.fori_loop(0, chunks, body, 0)

    return k


def kernel(token_ids, table):
    b0, b1 = token_ids.shape
    flat = token_ids.reshape(-1).astype(jnp.int32)
    idx2d = flat.reshape(-1, L)
    out = _make_sc_gather(idx2d.shape[0])(idx2d, table)
    return out.reshape(b0, b1, EMB_D)


# speedup vs baseline: 5.0257x; 1.0467x over previous
import functools

import jax
import jax.numpy as jnp
from jax import lax
from jax.experimental import pallas as pl
from jax.experimental.pallas import tpu as pltpu
from jax.experimental.pallas import tpu_sc as plsc

EMB_D = 32
L = 128
CR = 8
NW = 32


@functools.cache
def _make_sc_gather(n_rows_idx: int):
    assert n_rows_idx % (NW * CR) == 0
    rows_per_w = n_rows_idx // NW
    chunks = rows_per_w // CR
    assert chunks % 2 == 0
    mesh = plsc.VectorSubcoreMesh(core_axis_name="c", subcore_axis_name="s")

    @functools.partial(
        pl.kernel,
        mesh=mesh,
        out_type=jax.ShapeDtypeStruct((n_rows_idx * L, EMB_D), jnp.float32),
        scratch_types=[
            pltpu.VMEM((2, CR, L), jnp.int32),
            pltpu.VMEM((2, CR * L, EMB_D), jnp.float32),
            pltpu.SemaphoreType.DMA,
            pltpu.SemaphoreType.DMA,
            pltpu.SemaphoreType.DMA,
            pltpu.SemaphoreType.DMA,
            pltpu.SemaphoreType.DMA,
            pltpu.SemaphoreType.DMA,
        ],
        compiler_params=pltpu.CompilerParams(use_tc_tiling_on_sc=False),
    )
    def k(idx_hbm, table_hbm, out_hbm, idx_v, rows_v, sa0, sa1, sb0, sb1,
          sc0, sc1):
        wid = lax.axis_index("s") * 2 + lax.axis_index("c")
        row_base = wid * rows_per_w
        sa, sb, sc = (sa0, sa1), (sb0, sb1), (sc0, sc1)

        def idx_src(g):
            return idx_hbm.at[pl.ds(row_base + g * CR, CR), :]

        def out_dst(g):
            return out_hbm.at[pl.ds((row_base + g * CR) * L, CR * L), :]

        pltpu.async_copy(idx_src(0), idx_v.at[0], sa[0])
        pltpu.async_copy(idx_src(1), idx_v.at[1], sa[1])

        def body(i, carry):
            for b in (0, 1):
                g = i * 2 + b
                pltpu.make_async_copy(idx_src(g), idx_v.at[b], sa[b]).wait()

                @pl.when(g >= 2)
                def _wait_store():
                    pltpu.make_async_copy(
                        rows_v.at[b], out_dst(g - 2), sc[b]).wait()

                copies = [
                    pltpu.async_copy(
                        table_hbm.at[idx_v.at[b, j]],
                        rows_v.at[b, pl.ds(j * L, L), :],
                        sb[b],
                    )
                    for j in range(CR)
                ]
                for c in copies:
                    c.wait()

                @pl.when(g + 2 < chunks)
                def _prefetch_idx():
                    pltpu.async_copy(idx_src(g + 2), idx_v.at[b], sa[b])

                pltpu.async_copy(rows_v.at[b], out_dst(g), sc[b])
            return carry

        lax.fori_loop(0, chunks // 2, body, 0)

        for b in (0, 1):
            pltpu.make_async_copy(
                rows_v.at[b], out_dst(chunks - 2 + b), sc[b]).wait()

    return k


def kernel(token_ids, table):
    b0, b1 = token_ids.shape
    flat = token_ids.reshape(-1).astype(jnp.int32)
    idx2d = flat.reshape(-1, L)
    out = _make_sc_gather(idx2d.shape[0])(idx2d, table)
    return out.reshape(b0, b1, EMB_D)
